# Initial kernel scaffold; baseline (speedup 1.0000x reference)
import jax
import jax.numpy as jnp
from jax.experimental import pallas as pl


def kernel(input, weight):
    raise NotImplementedError("write your pallas kernel here")


# speedup vs baseline: 1.8748x; 1.8748x over previous
import functools

import jax
import jax.numpy as jnp
from jax import lax
from jax.experimental import pallas as pl
from jax.experimental.pallas import tpu as pltpu
from jax.experimental.pallas import tpu_sc as plsc

VOCAB = 1_000_000
EMB = 64
BATCH = 16384
SEQ = 50
N = BATCH * SEQ

NC = 2
NS = 16
NW = NC * NS
NPW = N // NW
CH = 128
NCH = NPW // CH
NBUF = 8
DELAY = 4
assert NCH % NBUF == 0


def _body(idx_hbm, w_hbm, out_hbm, idx_v, rows_v, gsem, ssem):
    wid = lax.axis_index("s") * NC + lax.axis_index("c")
    row0 = wid * NPW

    pltpu.sync_copy(idx_hbm.at[wid], idx_v)

    def gather_desc(chunk, buf):
        return pltpu.make_async_copy(
            w_hbm.at[idx_v.at[chunk]], rows_v.at[buf], gsem.at[buf])

    def store_desc(chunk, buf):
        return pltpu.make_async_copy(
            rows_v.at[buf], out_hbm.at[pl.ds(row0 + chunk * CH, CH)],
            ssem.at[buf])

    for c in range(DELAY):
        gather_desc(c, c).start()

    @pl.loop(0, NCH // NBUF)
    def _(g):
        for b in range(NBUF):
            t = g * NBUF + b
            bg = (b + DELAY) % NBUF

            @pl.when(t + DELAY < NCH)
            def _():
                @pl.when(t >= NBUF - DELAY)
                def _():
                    store_desc(t + DELAY - NBUF, bg).wait()
                gather_desc(t + DELAY, bg).start()

            gather_desc(t, b).wait()
            store_desc(t, b).start()

    for b in range(NBUF):
        store_desc(NCH - NBUF + b, b).wait()


@jax.jit
def _gather(idx, weight):
    mesh = plsc.VectorSubcoreMesh(core_axis_name="c", subcore_axis_name="s")
    return pl.kernel(
        _body,
        out_type=jax.ShapeDtypeStruct((N, EMB), jnp.float32),
        mesh=mesh,
        scratch_types=[
            pltpu.VMEM((NCH, CH), jnp.int32),
            pltpu.VMEM((NBUF, CH, EMB), jnp.float32),
            pltpu.SemaphoreType.DMA((NBUF,)),
            pltpu.SemaphoreType.DMA((NBUF,)),
        ],
        compiler_params=pltpu.CompilerParams(use_tc_tiling_on_sc=False),
    )(idx, weight)


def kernel(input, weight):
    idx = input.reshape(NW, NCH, CH).astype(jnp.int32)
    out = _gather(idx, weight)
    return out.reshape(BATCH, SEQ, EMB)
